# baseline (speedup 1.0000x reference)
import jax
import jax.numpy as jnp
from jax import lax
from jax.experimental import pallas as pl
from jax.experimental.pallas import tpu as pltpu


def _gate_kernel(xa_ref, xb_ref, w_ref, b_ref, o_ref):
    w = w_ref[...]
    bb = b_ref[...]
    for half, x_ref in ((0, xa_ref), (1, xb_ref)):
        x = x_ref[...]
        logits = lax.dot_general(
            x, w, (((1,), (1,)), ((), ())), preferred_element_type=jnp.float32
        )
        logits = logits + bb
        m = jnp.max(logits, axis=-1, keepdims=True)
        e = jnp.exp(logits - m)
        n = x.shape[0]
        o_ref[pl.ds(half * n, n), :] = e / jnp.sum(e, axis=-1, keepdims=True)


def kernel(x, W, b):
    T, D = x.shape
    E = W.shape[0]
    BT = 512
    b2 = b.reshape(1, E)
    return pl.pallas_call(
        _gate_kernel,
        grid=(T // (2 * BT),),
        in_specs=[
            pl.BlockSpec((BT, D), lambda i: (2 * i, 0)),
            pl.BlockSpec((BT, D), lambda i: (2 * i + 1, 0)),
            pl.BlockSpec((E, D), lambda i: (0, 0)),
            pl.BlockSpec((1, E), lambda i: (0, 0)),
        ],
        out_specs=pl.BlockSpec((2 * BT, E), lambda i: (i, 0)),
        out_shape=jax.ShapeDtypeStruct((T, E), jnp.float32),
        compiler_params=pltpu.CompilerParams(
            dimension_semantics=("parallel",),
        ),
    )(x, x, W, b2)


# speedup vs baseline: 1.0105x; 1.0105x over previous
import jax
import jax.numpy as jnp
from jax import lax
from jax.experimental import pallas as pl
from jax.experimental.pallas import tpu as pltpu

_K = 4


def _gate_kernel(x_hbm, w_ref, b_ref, o_ref, buf, sem):
    i = pl.program_id(0)
    nb = pl.num_programs(0)
    bt = buf.shape[1]

    def start(chunk, slot):
        pltpu.make_async_copy(
            x_hbm.at[pl.ds(chunk * bt, bt), :], buf.at[slot], sem.at[slot]
        ).start()

    @pl.when(i == 0)
    def _():
        for k in range(_K):
            start(k, k)

    @pl.when(jnp.logical_and(i > 0, i + _K - 1 < nb))
    def _():
        chunk = i + _K - 1
        start(chunk, lax.rem(chunk, _K))

    slot = lax.rem(i, _K)
    pltpu.make_async_copy(
        x_hbm.at[pl.ds(i * bt, bt), :], buf.at[slot], sem.at[slot]
    ).wait()

    x = buf[slot]
    logits = lax.dot_general(
        x, w_ref[...], (((1,), (1,)), ((), ())), preferred_element_type=jnp.float32
    )
    logits = logits + b_ref[...]
    m = jnp.max(logits, axis=-1, keepdims=True)
    e = jnp.exp(logits - m)
    o_ref[...] = e / jnp.sum(e, axis=-1, keepdims=True)


def kernel(x, W, b):
    T, D = x.shape
    E = W.shape[0]
    BT = 512
    b2 = b.reshape(1, E)
    return pl.pallas_call(
        _gate_kernel,
        grid=(T // BT,),
        in_specs=[
            pl.BlockSpec(memory_space=pl.MemorySpace.ANY),
            pl.BlockSpec((E, D), lambda i: (0, 0)),
            pl.BlockSpec((1, E), lambda i: (0, 0)),
        ],
        out_specs=pl.BlockSpec((BT, E), lambda i: (i, 0)),
        out_shape=jax.ShapeDtypeStruct((T, E), jnp.float32),
        scratch_shapes=[
            pltpu.VMEM((_K, BT, D), jnp.float32),
            pltpu.SemaphoreType.DMA((_K,)),
        ],
        compiler_params=pltpu.CompilerParams(
            dimension_semantics=("arbitrary",),
        ),
    )(x, W, b2)
